# baseline (speedup 1.0000x reference)
import functools

import jax
import jax.numpy as jnp
from jax import lax
from jax.experimental import pallas as pl
from jax.experimental.pallas import tpu as pltpu
from jax.experimental.pallas import tpu_sc as plsc

NC = 2
NS = 16
NW = NC * NS
CHUNK = 128
NBUF = 2


@functools.lru_cache(maxsize=None)
def _build(N, DP, n_chunks):
    per_w = n_chunks * CHUNK
    mesh = plsc.VectorSubcoreMesh(core_axis_name="c", subcore_axis_name="s")

    scratch = [
        pltpu.VMEM((n_chunks, CHUNK), jnp.int32),
        pltpu.VMEM((NBUF, CHUNK, DP), jnp.float32),
    ] + [pltpu.SemaphoreType.DMA] * (2 * NBUF)

    @functools.partial(
        pl.kernel,
        out_type=jax.ShapeDtypeStruct((N, DP), jnp.float32),
        mesh=mesh,
        scratch_types=scratch,
        compiler_params=pltpu.CompilerParams(use_tc_tiling_on_sc=False),
    )
    def run(idx_hbm, table_hbm, out_hbm, idx_v, rows_v, *sems):
        gsem = sems[:NBUF]
        ssem = sems[NBUF:]
        wid = lax.axis_index("s") * NC + lax.axis_index("c")
        base = wid * per_w
        pltpu.sync_copy(idx_hbm.at[wid], idx_v)

        def gather_start(c, b):
            pltpu.async_copy(table_hbm.at[idx_v.at[c]], rows_v.at[b], gsem[b])

        def gather_wait(c, b):
            pltpu.make_async_copy(
                table_hbm.at[idx_v.at[c]], rows_v.at[b], gsem[b]).wait()

        def scatter_start(c, b):
            pltpu.async_copy(
                rows_v.at[b], out_hbm.at[pl.ds(base + c * CHUNK, CHUNK)], ssem[b])

        def scatter_wait(c, b):
            pltpu.make_async_copy(
                rows_v.at[b], out_hbm.at[pl.ds(base + c * CHUNK, CHUNK)], ssem[b]).wait()

        for b in range(NBUF):
            gather_start(b, b)

        n_outer = n_chunks // NBUF

        @pl.loop(0, n_outer - 1)
        def _(o):
            for b in range(NBUF):
                c = o * NBUF + b
                gather_wait(c, b)
                scatter_start(c, b)
                scatter_wait(c, b)
                gather_start(c + NBUF, b)

        for b in range(NBUF):
            c = (n_outer - 1) * NBUF + b
            gather_wait(c, b)
            scatter_start(c, b)
            scatter_wait(c, b)

    return run


def kernel(inputs, word_embedding):
    B, L = inputs.shape
    V, D = word_embedding.shape
    DP = (D + 7) // 8 * 8
    N = B * L
    per_w = N // NW
    n_chunks = per_w // CHUNK
    idx = inputs.reshape(NW, n_chunks, CHUNK).astype(jnp.int32)
    table = jnp.pad(word_embedding, ((0, 0), (0, DP - D)))
    out = _build(N, DP, n_chunks)(idx, table)
    return out[:, :D].reshape(B, L, D)
